# baseline (speedup 1.0000x reference)
import functools

import jax
import jax.numpy as jnp
from jax import lax
from jax.experimental import pallas as pl
from jax.experimental.pallas import tpu as pltpu
from jax.experimental.pallas import tpu_sc as plsc

H = 128
F = 4096
N = 16384
E = 262144
NC = 2
NS = 16
NW = NC * NS

CC = 128
BB = 256
BSH = 8
SUB = 128
CAP = 9216
NBMAX = CAP // BB
SEG = 2048
EPT = E // NS
HALF = N // 2
ACCR = HALF + 8
RZ = HALF // NS


def _sc_mesh():
    return plsc.VectorSubcoreMesh(core_axis_name="c", subcore_axis_name="s")


def _make_deg_kernel():
    EPW = E // NW
    RR = N // NS

    @functools.partial(
        pl.kernel,
        mesh=_sc_mesh(),
        out_type=jax.ShapeDtypeStruct((NC * N,), jnp.float32),
        scratch_types=[
            pltpu.VMEM((EPW,), jnp.int32),
            pltpu.VMEM((N,), jnp.float32),
            pltpu.VMEM((NS, RR), jnp.float32),
            pltpu.VMEM((RR,), jnp.float32),
            pltpu.VMEM_SHARED((NS, N), jnp.float32),
        ],
        compiler_params=pltpu.CompilerParams(needs_layout_passes=False),
    )
    def deg_kernel(dst_hbm, zeros_hbm, out_hbm, didx, hist, tmp16, accv, stage):
        cid = lax.axis_index("c")
        sid = lax.axis_index("s")
        wid = cid * NS + sid
        pltpu.sync_copy(zeros_hbm, hist)
        pltpu.sync_copy(dst_hbm.at[pl.ds(wid * EPW, EPW)], didx)
        ones = jnp.ones((16,), jnp.float32)

        def hbody(j, carry):
            dvec = didx[pl.ds(j * 16, 16)]
            plsc.addupdate_scatter(hist, [dvec], ones)
            return carry

        lax.fori_loop(0, EPW // 16, hbody, 0)
        pltpu.sync_copy(hist, stage.at[sid])
        plsc.subcore_barrier()
        pltpu.sync_copy(stage.at[:, pl.ds(sid * RR, RR)], tmp16)

        def rbody(j, carry):
            s = tmp16[0, pl.ds(j * 16, 16)]
            for k in range(1, NS):
                s = s + tmp16[k, pl.ds(j * 16, 16)]
            accv[pl.ds(j * 16, 16)] = s
            return carry

        lax.fori_loop(0, RR // 16, rbody, 0)
        pltpu.sync_copy(accv, out_hbm.at[pl.ds(cid * N + sid * RR, RR)])

    return deg_kernel


def _make_scatter_kernel(nchunk):
    @functools.partial(
        pl.kernel,
        mesh=_sc_mesh(),
        out_type=tuple(
            jax.ShapeDtypeStruct((N, CC), jnp.float32) for _ in range(nchunk)
        ),
        scratch_types=[
            pltpu.VMEM((SEG,), jnp.int32),
            pltpu.VMEM((SEG,), jnp.int32),
            pltpu.VMEM((CAP,), jnp.int32),
            pltpu.VMEM((CAP,), jnp.int32),
            pltpu.VMEM((SUB,), jnp.int32),
            pltpu.VMEM((SUB,), jnp.int32),
            pltpu.VMEM((BB,), jnp.int32),
            pltpu.VMEM((BB, CC), jnp.float32),
            pltpu.VMEM_SHARED((ACCR, CC), jnp.float32),
            pltpu.SemaphoreType.DMA,
            pltpu.SemaphoreType.DMA,
        ],
        compiler_params=pltpu.CompilerParams(needs_layout_passes=False),
    )
    def scatter_kernel(*refs):
        tables = refs[:nchunk]
        src_hbm, dst_hbm, zrows = refs[nchunk:nchunk + 3]
        outs = refs[nchunk + 3:2 * nchunk + 3]
        (sraw, draw, sflat, dflat, sidxl, sidxh, didx,
         msg, acc, semg, sems) = refs[2 * nchunk + 3:]
        cid = lax.axis_index("c")
        sid = lax.axis_index("s")
        base_node = cid * HALF
        ebase = sid * EPT
        iota = lax.iota(jnp.int32, 16)

        def seg_body(g, cnt):
            pltpu.sync_copy(src_hbm.at[pl.ds(ebase + g * SEG, SEG)], sraw)
            pltpu.sync_copy(dst_hbm.at[pl.ds(ebase + g * SEG, SEG)], draw)

            def vec_body(j, cnt_vec):
                dloc = draw[pl.ds(j * 16, 16)] - base_node
                svec = sraw[pl.ds(j * 16, 16)]
                m = (dloc >= 0) & (dloc < HALF)
                mi = m.astype(jnp.int32)
                pos = cnt_vec + plsc.cumsum(mi) - 1
                m = m & (pos < CAP)
                plsc.store_scatter(dflat, [pos], dloc, mask=m)
                plsc.store_scatter(sflat, [pos], svec, mask=m)
                return cnt_vec + plsc.all_reduce_population_count(m)

            return lax.fori_loop(0, SEG // 16, vec_body, cnt)

        cnt_vec = lax.fori_loop(0, EPT // SEG, seg_body,
                                jnp.zeros((16,), jnp.int32))
        cnt = jnp.max(cnt_vec)

        pad = jnp.bitwise_and(-cnt, 2 * BB - 1)
        dumpd = jnp.full((16,), HALF, jnp.int32)
        zeros16 = jnp.zeros((16,), jnp.int32)
        for j in range(2 * BB // 16):
            off = j * 16 + iota
            m = off < pad
            pos = cnt + off
            plsc.store_scatter(dflat, [pos], dumpd, mask=m)
            plsc.store_scatter(sflat, [pos], zeros16, mask=m)
        nb = jnp.right_shift(cnt + pad, BSH)

        for k in range(nchunk):
            pltpu.sync_copy(zrows, acc.at[pl.ds(sid * RZ, RZ)])

            @pl.when(sid == 0)
            def _():
                pltpu.sync_copy(zrows.at[pl.ds(0, 8)], acc.at[pl.ds(HALF, 8)])

            plsc.subcore_barrier()

            def body(t, carry):
                base = t * BB
                for i in range(SUB // 16):
                    sidxl[pl.ds(i * 16, 16)] = sflat[pl.ds(base + i * 16, 16)]
                for i in range(SUB // 16):
                    sidxh[pl.ds(i * 16, 16)] = \
                        sflat[pl.ds(base + SUB + i * 16, 16)]
                for i in range(BB // 16):
                    didx[pl.ds(i * 16, 16)] = dflat[pl.ds(base + i * 16, 16)]
                gl = pltpu.async_copy(tables[k].at[sidxl],
                                      msg.at[pl.ds(0, SUB)], semg)
                gh = pltpu.async_copy(tables[k].at[sidxh],
                                      msg.at[pl.ds(SUB, SUB)], semg)
                gl.wait()
                gh.wait()
                pltpu.async_copy(msg, acc.at[didx], sems, add=True).wait()
                return carry

            lax.fori_loop(0, nb, body, 0)
            plsc.subcore_barrier()
            pltpu.sync_copy(
                acc.at[pl.ds(sid * RZ, RZ)],
                outs[k].at[pl.ds(base_node + sid * RZ, RZ)],
            )

    return scatter_kernel


RB = 512
KB = 512
KS = F // KB


def _mm_body(x_ref, w_ref, h_ref, w1h_ref, dis_ref,
             pa, pb, m2_ref, acc_ref):
    k = pl.program_id(1)

    @pl.when(k == 0)
    def _():
        acc_ref[...] = jnp.zeros_like(acc_ref)

    acc_ref[...] += jnp.dot(x_ref[...].astype(jnp.bfloat16), w_ref[...],
                            preferred_element_type=jnp.float32)

    @pl.when(k == KS - 1)
    def _():
        dis = dis_ref[...]
        m1 = acc_ref[:, :2 * H] + jnp.dot(h_ref[...], w1h_ref[...],
                                          preferred_element_type=jnp.float32)
        ps1 = m1 * dis
        pa[...] = ps1[:, :H]
        pb[...] = ps1[:, H:]
        m2_ref[...] = acc_ref[:, 2 * H:] * dis


def _mm_call(x, wcat, h, w1h, dis):
    return pl.pallas_call(
        _mm_body,
        grid=(N // RB, KS),
        in_specs=[
            pl.BlockSpec((RB, KB), lambda i, k: (i, k)),
            pl.BlockSpec((KB, 3 * H), lambda i, k: (k, 0)),
            pl.BlockSpec((RB, H), lambda i, k: (i, 0)),
            pl.BlockSpec((H, 2 * H), lambda i, k: (0, 0)),
            pl.BlockSpec((RB, 1), lambda i, k: (i, 0)),
        ],
        out_specs=[pl.BlockSpec((RB, H), lambda i, k: (i, 0))] * 3,
        out_shape=[jax.ShapeDtypeStruct((N, H), jnp.float32)] * 3,
        scratch_shapes=[pltpu.VMEM((RB, 3 * H), jnp.float32)],
        compiler_params=pltpu.CompilerParams(
            dimension_semantics=("parallel", "arbitrary")),
    )(x, wcat, h, w1h, dis)


RG = 512


def _gate1_body(sa, sb, pa, pb, dis_ref, b1_ref, ru_ref):
    dis = dis_ref[...]
    ss = (sa, sb)
    ps = (pa, pb)
    for k in range(2):
        q = (ss[k][...] + ps[k][...]) * dis
        q = q + b1_ref[0, k * H:(k + 1) * H][None, :]
        ru_ref[:, k * H:(k + 1) * H] = jax.nn.sigmoid(q)


def _gate1_call(s1a, s1b, pa, pb, dis, b1r):
    blk = pl.BlockSpec((RG, H), lambda i: (i, 0))
    return pl.pallas_call(
        _gate1_body,
        grid=(N // RG,),
        in_specs=[blk] * 4
        + [pl.BlockSpec((RG, 1), lambda i: (i, 0)),
           pl.BlockSpec((1, 2 * H), lambda i: (0, 0))],
        out_specs=pl.BlockSpec((RG, 2 * H), lambda i: (i, 0)),
        out_shape=jax.ShapeDtypeStruct((N, 2 * H), jnp.float32),
        compiler_params=pltpu.CompilerParams(
            dimension_semantics=("parallel",)),
    )(s1a, s1b, pa, pb, dis, b1r)


def _mm2_body(r_ref, h_ref, w2h_ref, m2s_ref, dis_ref, q_ref):
    rh = r_ref[...] * h_ref[...]
    prod = jnp.dot(rh, w2h_ref[...], preferred_element_type=jnp.float32)
    q_ref[...] = m2s_ref[...] + prod * dis_ref[...]


def _mm2_call(r, h, w2h, m2s, dis):
    return pl.pallas_call(
        _mm2_body,
        grid=(N // RG,),
        in_specs=[
            pl.BlockSpec((RG, H), lambda i: (i, 0)),
            pl.BlockSpec((RG, H), lambda i: (i, 0)),
            pl.BlockSpec((H, H), lambda i: (0, 0)),
            pl.BlockSpec((RG, H), lambda i: (i, 0)),
            pl.BlockSpec((RG, 1), lambda i: (i, 0)),
        ],
        out_specs=pl.BlockSpec((RG, H), lambda i: (i, 0)),
        out_shape=jax.ShapeDtypeStruct((N, H), jnp.float32),
        compiler_params=pltpu.CompilerParams(
            dimension_semantics=("parallel",)),
    )(r, h, w2h, m2s, dis)


def _gate2_body(s2, p2, dis_ref, b2_ref, u_ref, h_ref, out_ref):
    q = (s2[...] + p2[...]) * dis_ref[...] + b2_ref[0, :][None, :]
    cv = jnp.tanh(q)
    u = u_ref[...]
    out_ref[...] = u * h_ref[...] + (1.0 - u) * cv


def _gate2_call(s2, ps2, dis, b2r, u, h):
    blk = pl.BlockSpec((RG, H), lambda i: (i, 0))
    return pl.pallas_call(
        _gate2_body,
        grid=(N // RG,),
        in_specs=[blk, blk,
                  pl.BlockSpec((RG, 1), lambda i: (i, 0)),
                  pl.BlockSpec((1, H), lambda i: (0, 0)),
                  blk, blk],
        out_specs=blk,
        out_shape=jax.ShapeDtypeStruct((N, H), jnp.float32),
        compiler_params=pltpu.CompilerParams(
            dimension_semantics=("parallel",)),
    )(s2, ps2, dis, b2r, u, h)


_deg_call = _make_deg_kernel()
_scatter_conv1 = _make_scatter_kernel(2)
_scatter_conv2 = _make_scatter_kernel(1)


def kernel(x, edge_index, hidden_state, W1, b1, W2, b2):
    src = edge_index[0]
    dst = edge_index[1]
    W1x, W1h = W1[:F], W1[F:]
    W2x, W2h = W2[:F], W2[F:]
    wcat = jnp.concatenate([W1x, W2x], axis=1).astype(jnp.bfloat16)
    zeros_n = jnp.zeros((N,), jnp.float32)
    zrows = jnp.zeros((RZ, CC), jnp.float32)

    degp = _deg_call(dst, zeros_n)
    deg = 1.0 + degp[:N] + degp[N:]
    dis = lax.rsqrt(deg).reshape(N, 1)

    ps1a, ps1b, m2s = _mm_call(x, wcat, hidden_state, W1h, dis)

    s1a, s1b = _scatter_conv1(ps1a, ps1b, src, dst, zrows)

    ru = _gate1_call(s1a, s1b, ps1a, ps1b, dis, b1.reshape(1, 2 * H))

    ru3 = ru.reshape(N // F, 2, (F // 2) * 2 * H)
    r = ru3[:, 0].reshape(N, H)
    u = ru3[:, 1].reshape(N, H)

    ps2 = _mm2_call(r, hidden_state, W2h, m2s, dis)
    (s2,) = _scatter_conv2(ps2, src, dst, zrows)

    return _gate2_call(s2, ps2, dis, b2.reshape(1, H), u, hidden_state)


# speedup vs baseline: 1.7510x; 1.7510x over previous
import functools

import jax
import jax.numpy as jnp
from jax import lax
from jax.experimental import pallas as pl
from jax.experimental.pallas import tpu as pltpu
from jax.experimental.pallas import tpu_sc as plsc

H = 128
F = 4096
N = 16384
E = 262144
NC = 2
NS = 16
NW = NC * NS

CC = 64
BB = 512


def _sc_mesh():
    return plsc.VectorSubcoreMesh(core_axis_name="c", subcore_axis_name="s")


def _make_deg_kernel():
    EPW = E // NW
    RR = N // NS

    @functools.partial(
        pl.kernel,
        mesh=_sc_mesh(),
        out_type=jax.ShapeDtypeStruct((NC * N,), jnp.float32),
        scratch_types=[
            pltpu.VMEM((EPW,), jnp.int32),
            pltpu.VMEM((N,), jnp.float32),
            pltpu.VMEM((NS, RR), jnp.float32),
            pltpu.VMEM((RR,), jnp.float32),
            pltpu.VMEM_SHARED((NS, N), jnp.float32),
        ],
        compiler_params=pltpu.CompilerParams(needs_layout_passes=False),
    )
    def deg_kernel(dst_hbm, zeros_hbm, out_hbm, didx, hist, tmp16, accv, stage):
        cid = lax.axis_index("c")
        sid = lax.axis_index("s")
        wid = cid * NS + sid
        pltpu.sync_copy(zeros_hbm, hist)
        pltpu.sync_copy(dst_hbm.at[pl.ds(wid * EPW, EPW)], didx)
        ones = jnp.ones((16,), jnp.float32)

        def hbody(j, carry):
            dvec = didx[pl.ds(j * 16, 16)]
            plsc.addupdate_scatter(hist, [dvec], ones)
            return carry

        lax.fori_loop(0, EPW // 16, hbody, 0)
        pltpu.sync_copy(hist, stage.at[sid])
        plsc.subcore_barrier()
        pltpu.sync_copy(stage.at[:, pl.ds(sid * RR, RR)], tmp16)

        def rbody(j, carry):
            s = tmp16[0, pl.ds(j * 16, 16)]
            for k in range(1, NS):
                s = s + tmp16[k, pl.ds(j * 16, 16)]
            accv[pl.ds(j * 16, 16)] = s
            return carry

        lax.fori_loop(0, RR // 16, rbody, 0)
        pltpu.sync_copy(accv, out_hbm.at[pl.ds(cid * N + sid * RR, RR)])

    return deg_kernel


def _make_scatter_kernel(nchunk):
    EPW = E // NW
    RZ = N // NS
    NB = EPW // BB

    @functools.partial(
        pl.kernel,
        mesh=_sc_mesh(),
        out_type=tuple(
            jax.ShapeDtypeStruct((NC * N, CC), jnp.float32)
            for _ in range(nchunk)
        ),
        scratch_types=[
            pltpu.VMEM((EPW,), jnp.int32),
            pltpu.VMEM((EPW,), jnp.int32),
            pltpu.VMEM((BB,), jnp.int32),
            pltpu.VMEM((BB,), jnp.int32),
            pltpu.VMEM((BB, CC), jnp.float32),
            pltpu.VMEM_SHARED((N, CC), jnp.float32),
            pltpu.SemaphoreType.DMA,
        ],
        compiler_params=pltpu.CompilerParams(use_tc_tiling_on_sc=False),
    )
    def scatter_kernel(*refs):
        tables = refs[:nchunk]
        src_hbm, dst_hbm, zrows = refs[nchunk:nchunk + 3]
        outs = refs[nchunk + 3:2 * nchunk + 3]
        sall, dall, sidx, didx, msg, acc, sem = refs[2 * nchunk + 3:]
        cid = lax.axis_index("c")
        sid = lax.axis_index("s")
        ebase = (cid * NS + sid) * EPW
        pltpu.sync_copy(src_hbm.at[pl.ds(ebase, EPW)], sall)
        pltpu.sync_copy(dst_hbm.at[pl.ds(ebase, EPW)], dall)
        for c in range(nchunk):
            pltpu.sync_copy(zrows, acc.at[pl.ds(sid * RZ, RZ)])
            plsc.subcore_barrier()
            for b in range(NB):
                base = b * BB
                for i in range(BB // 16):
                    sidx[pl.ds(i * 16, 16)] = sall[pl.ds(base + i * 16, 16)]
                    didx[pl.ds(i * 16, 16)] = dall[pl.ds(base + i * 16, 16)]
                pltpu.async_copy(tables[c].at[sidx], msg, sem).wait()
                pltpu.sync_copy(msg, acc.at[didx], add=True)
            plsc.subcore_barrier()
            pltpu.sync_copy(
                acc.at[pl.ds(sid * RZ, RZ)],
                outs[c].at[pl.ds(cid * N + sid * RZ, RZ)],
            )

    return scatter_kernel


RB = 512
KB = 512
KS = F // KB


def _mm_body(x_ref, w_ref, h_ref, w1h_ref, dis_ref,
             p0, p1, p2, p3, m2_ref, acc_ref):
    k = pl.program_id(1)

    @pl.when(k == 0)
    def _():
        acc_ref[...] = jnp.zeros_like(acc_ref)

    acc_ref[...] += jnp.dot(x_ref[...].astype(jnp.bfloat16), w_ref[...],
                            preferred_element_type=jnp.float32)

    @pl.when(k == KS - 1)
    def _():
        dis = dis_ref[...]
        m1 = acc_ref[:, :2 * H] + jnp.dot(h_ref[...], w1h_ref[...],
                                          preferred_element_type=jnp.float32)
        ps1 = m1 * dis
        p0[...] = ps1[:, 0:64]
        p1[...] = ps1[:, 64:128]
        p2[...] = ps1[:, 128:192]
        p3[...] = ps1[:, 192:256]
        m2_ref[...] = acc_ref[:, 2 * H:] * dis


def _mm_call(x, wcat, h, w1h, dis):
    return pl.pallas_call(
        _mm_body,
        grid=(N // RB, KS),
        in_specs=[
            pl.BlockSpec((RB, KB), lambda i, k: (i, k)),
            pl.BlockSpec((KB, 3 * H), lambda i, k: (k, 0)),
            pl.BlockSpec((RB, H), lambda i, k: (i, 0)),
            pl.BlockSpec((H, 2 * H), lambda i, k: (0, 0)),
            pl.BlockSpec((RB, 1), lambda i, k: (i, 0)),
        ],
        out_specs=[pl.BlockSpec((RB, 64), lambda i, k: (i, 0))] * 4
        + [pl.BlockSpec((RB, H), lambda i, k: (i, 0))],
        out_shape=[jax.ShapeDtypeStruct((N, 64), jnp.float32)] * 4
        + [jax.ShapeDtypeStruct((N, H), jnp.float32)],
        scratch_shapes=[pltpu.VMEM((RB, 3 * H), jnp.float32)],
        compiler_params=pltpu.CompilerParams(
            dimension_semantics=("parallel", "arbitrary")),
    )(x, wcat, h, w1h, dis)


RG = 512


def _gate1_body(sa0, sa1, sa2, sa3, sb0, sb1, sb2, sb3,
                p0, p1, p2, p3, dis_ref, b1_ref, ru_ref):
    dis = dis_ref[...]
    sas = (sa0, sa1, sa2, sa3)
    sbs = (sb0, sb1, sb2, sb3)
    ps = (p0, p1, p2, p3)
    for c in range(4):
        q = (sas[c][...] + sbs[c][...] + ps[c][...]) * dis
        q = q + b1_ref[0, c * 64:(c + 1) * 64][None, :]
        ru_ref[:, c * 64:(c + 1) * 64] = jax.nn.sigmoid(q)


def _gate1_call(s1, ps1, dis, b1r):
    blk = pl.BlockSpec((RG, 64), lambda i: (i, 0))
    blk_hi = pl.BlockSpec((RG, 64), lambda i: (i + N // RG, 0))
    return pl.pallas_call(
        _gate1_body,
        grid=(N // RG,),
        in_specs=[blk] * 4 + [blk_hi] * 4 + [blk] * 4
        + [pl.BlockSpec((RG, 1), lambda i: (i, 0)),
           pl.BlockSpec((1, 2 * H), lambda i: (0, 0))],
        out_specs=pl.BlockSpec((RG, 2 * H), lambda i: (i, 0)),
        out_shape=jax.ShapeDtypeStruct((N, 2 * H), jnp.float32),
        compiler_params=pltpu.CompilerParams(
            dimension_semantics=("parallel",)),
    )(*s1, *s1, *ps1, dis, b1r)


def _mm2_body(r_ref, h_ref, w2h_ref, m2s_ref, dis_ref, q0_ref, q1_ref):
    rh = r_ref[...] * h_ref[...]
    prod = jnp.dot(rh, w2h_ref[...], preferred_element_type=jnp.float32)
    ps2 = m2s_ref[...] + prod * dis_ref[...]
    q0_ref[...] = ps2[:, :64]
    q1_ref[...] = ps2[:, 64:]


def _mm2_call(r, h, w2h, m2s, dis):
    return pl.pallas_call(
        _mm2_body,
        grid=(N // RG,),
        in_specs=[
            pl.BlockSpec((RG, H), lambda i: (i, 0)),
            pl.BlockSpec((RG, H), lambda i: (i, 0)),
            pl.BlockSpec((H, H), lambda i: (0, 0)),
            pl.BlockSpec((RG, H), lambda i: (i, 0)),
            pl.BlockSpec((RG, 1), lambda i: (i, 0)),
        ],
        out_specs=[pl.BlockSpec((RG, 64), lambda i: (i, 0))] * 2,
        out_shape=[jax.ShapeDtypeStruct((N, 64), jnp.float32)] * 2,
        compiler_params=pltpu.CompilerParams(
            dimension_semantics=("parallel",)),
    )(r, h, w2h, m2s, dis)


def _gate2_body(sa0, sa1, sb0, sb1, p0, p1, dis_ref, b2_ref,
                u_ref, h_ref, out_ref):
    dis = dis_ref[...]
    u = u_ref[...]
    h = h_ref[...]
    sas = (sa0, sa1)
    sbs = (sb0, sb1)
    ps = (p0, p1)
    for c in range(2):
        q = (sas[c][...] + sbs[c][...] + ps[c][...]) * dis
        q = q + b2_ref[0, c * 64:(c + 1) * 64][None, :]
        cv = jnp.tanh(q)
        lo, hi = c * 64, (c + 1) * 64
        out_ref[:, lo:hi] = u[:, lo:hi] * h[:, lo:hi] + (1.0 - u[:, lo:hi]) * cv


def _gate2_call(s2, ps2, dis, b2r, u, h):
    blk = pl.BlockSpec((RG, 64), lambda i: (i, 0))
    blk_hi = pl.BlockSpec((RG, 64), lambda i: (i + N // RG, 0))
    blkh = pl.BlockSpec((RG, H), lambda i: (i, 0))
    return pl.pallas_call(
        _gate2_body,
        grid=(N // RG,),
        in_specs=[blk] * 2 + [blk_hi] * 2 + [blk] * 2
        + [pl.BlockSpec((RG, 1), lambda i: (i, 0)),
           pl.BlockSpec((1, H), lambda i: (0, 0)),
           blkh, blkh],
        out_specs=pl.BlockSpec((RG, H), lambda i: (i, 0)),
        out_shape=jax.ShapeDtypeStruct((N, H), jnp.float32),
        compiler_params=pltpu.CompilerParams(
            dimension_semantics=("parallel",)),
    )(*s2, *s2, *ps2, dis, b2r, u, h)


_deg_call = _make_deg_kernel()
_scatter4_call = _make_scatter_kernel(4)
_scatter2_call = _make_scatter_kernel(2)


def kernel(x, edge_index, hidden_state, W1, b1, W2, b2):
    src = edge_index[0]
    dst = edge_index[1]
    W1x, W1h = W1[:F], W1[F:]
    W2x, W2h = W2[:F], W2[F:]
    wcat = jnp.concatenate([W1x, W2x], axis=1).astype(jnp.bfloat16)
    zeros_n = jnp.zeros((N,), jnp.float32)
    zrows = jnp.zeros((N // NS, CC), jnp.float32)

    degp = _deg_call(dst, zeros_n)
    deg = 1.0 + degp[:N] + degp[N:]
    dis = lax.rsqrt(deg).reshape(N, 1)

    ps1 = _mm_call(x, wcat, hidden_state, W1h, dis)
    ps1c, m2s = ps1[:4], ps1[4]

    s1 = _scatter4_call(*ps1c, src, dst, zrows)

    ru = _gate1_call(s1, ps1c, dis, b1.reshape(1, 2 * H))

    ru3 = ru.reshape(N // F, 2, (F // 2) * 2 * H)
    r = ru3[:, 0].reshape(N, H)
    u = ru3[:, 1].reshape(N, H)

    ps2c = _mm2_call(r, hidden_state, W2h, m2s, dis)
    s2 = _scatter2_call(*ps2c, src, dst, zrows)

    return _gate2_call(s2, ps2c, dis, b2.reshape(1, H), u, hidden_state)


# speedup vs baseline: 1.7722x; 1.0121x over previous
import functools

import jax
import jax.numpy as jnp
from jax import lax
from jax.experimental import pallas as pl
from jax.experimental.pallas import tpu as pltpu
from jax.experimental.pallas import tpu_sc as plsc

H = 128
F = 4096
N = 16384
E = 262144
NC = 2
NS = 16
NW = NC * NS

CC = 64
BB = 256


def _sc_mesh():
    return plsc.VectorSubcoreMesh(core_axis_name="c", subcore_axis_name="s")


def _make_deg_kernel():
    EPW = E // NW
    RR = N // NS

    @functools.partial(
        pl.kernel,
        mesh=_sc_mesh(),
        out_type=jax.ShapeDtypeStruct((NC * N,), jnp.float32),
        scratch_types=[
            pltpu.VMEM((EPW,), jnp.int32),
            pltpu.VMEM((N,), jnp.float32),
            pltpu.VMEM((NS, RR), jnp.float32),
            pltpu.VMEM((RR,), jnp.float32),
            pltpu.VMEM_SHARED((NS, N), jnp.float32),
        ],
        compiler_params=pltpu.CompilerParams(needs_layout_passes=False),
    )
    def deg_kernel(dst_hbm, zeros_hbm, out_hbm, didx, hist, tmp16, accv, stage):
        cid = lax.axis_index("c")
        sid = lax.axis_index("s")
        wid = cid * NS + sid
        pltpu.sync_copy(zeros_hbm, hist)
        pltpu.sync_copy(dst_hbm.at[pl.ds(wid * EPW, EPW)], didx)
        ones = jnp.ones((16,), jnp.float32)

        def hbody(j, carry):
            dvec = didx[pl.ds(j * 16, 16)]
            plsc.addupdate_scatter(hist, [dvec], ones)
            return carry

        lax.fori_loop(0, EPW // 16, hbody, 0)
        pltpu.sync_copy(hist, stage.at[sid])
        plsc.subcore_barrier()
        pltpu.sync_copy(stage.at[:, pl.ds(sid * RR, RR)], tmp16)

        def rbody(j, carry):
            s = tmp16[0, pl.ds(j * 16, 16)]
            for k in range(1, NS):
                s = s + tmp16[k, pl.ds(j * 16, 16)]
            accv[pl.ds(j * 16, 16)] = s
            return carry

        lax.fori_loop(0, RR // 16, rbody, 0)
        pltpu.sync_copy(accv, out_hbm.at[pl.ds(cid * N + sid * RR, RR)])

    return deg_kernel


def _make_scatter_kernel(nchunk):
    EPW = E // NW
    RZ = N // NS
    NB = EPW // BB

    @functools.partial(
        pl.kernel,
        mesh=_sc_mesh(),
        out_type=tuple(
            jax.ShapeDtypeStruct((NC * N, CC), jnp.float32)
            for _ in range(nchunk)
        ),
        scratch_types=[
            pltpu.VMEM((EPW,), jnp.int32),
            pltpu.VMEM((EPW,), jnp.int32),
            pltpu.VMEM((BB,), jnp.int32),
            pltpu.VMEM((BB,), jnp.int32),
            pltpu.VMEM((BB,), jnp.int32),
            pltpu.VMEM((BB,), jnp.int32),
            pltpu.VMEM((BB, CC), jnp.float32),
            pltpu.VMEM((BB, CC), jnp.float32),
            pltpu.VMEM_SHARED((N, CC), jnp.float32),
            pltpu.SemaphoreType.DMA,
            pltpu.SemaphoreType.DMA,
            pltpu.SemaphoreType.DMA,
        ],
        compiler_params=pltpu.CompilerParams(use_tc_tiling_on_sc=False),
    )
    def scatter_kernel(*refs):
        tables = refs[:nchunk]
        src_hbm, dst_hbm, zrows = refs[nchunk:nchunk + 3]
        outs = refs[nchunk + 3:2 * nchunk + 3]
        (sall, dall, sidxa, didxa, sidxb, didxb,
         msga, msgb, acc, sema, semb, sems) = refs[2 * nchunk + 3:]
        cid = lax.axis_index("c")
        sid = lax.axis_index("s")
        ebase = (cid * NS + sid) * EPW
        pltpu.sync_copy(src_hbm.at[pl.ds(ebase, EPW)], sall)
        pltpu.sync_copy(dst_hbm.at[pl.ds(ebase, EPW)], dall)

        def copy_idx(base, sdst, ddst):
            for i in range(BB // 16):
                sdst[pl.ds(i * 16, 16)] = sall[pl.ds(base + i * 16, 16)]
                ddst[pl.ds(i * 16, 16)] = dall[pl.ds(base + i * 16, 16)]

        for c in range(nchunk):
            pltpu.sync_copy(zrows, acc.at[pl.ds(sid * RZ, RZ)])
            plsc.subcore_barrier()

            def body2(t, carry):
                copy_idx(2 * t * BB, sidxa, didxa)
                ga = pltpu.async_copy(tables[c].at[sidxa], msga, sema)
                copy_idx((2 * t + 1) * BB, sidxb, didxb)
                gb = pltpu.async_copy(tables[c].at[sidxb], msgb, semb)
                ga.wait()
                sa = pltpu.async_copy(msga, acc.at[didxa], sems, add=True)
                gb.wait()
                sb = pltpu.async_copy(msgb, acc.at[didxb], sems, add=True)
                sa.wait()
                sb.wait()
                return carry

            lax.fori_loop(0, NB // 2, body2, 0)
            plsc.subcore_barrier()
            pltpu.sync_copy(
                acc.at[pl.ds(sid * RZ, RZ)],
                outs[c].at[pl.ds(cid * N + sid * RZ, RZ)],
            )

    return scatter_kernel


RB = 512
KB = 512
KS = F // KB


def _mm_body(x_ref, w_ref, h_ref, w1h_ref, dis_ref,
             p0, p1, p2, p3, m2_ref, acc_ref):
    k = pl.program_id(1)

    @pl.when(k == 0)
    def _():
        acc_ref[...] = jnp.zeros_like(acc_ref)

    acc_ref[...] += jnp.dot(x_ref[...].astype(jnp.bfloat16), w_ref[...],
                            preferred_element_type=jnp.float32)

    @pl.when(k == KS - 1)
    def _():
        dis = dis_ref[...]
        m1 = acc_ref[:, :2 * H] + jnp.dot(h_ref[...], w1h_ref[...],
                                          preferred_element_type=jnp.float32)
        ps1 = m1 * dis
        p0[...] = ps1[:, 0:64]
        p1[...] = ps1[:, 64:128]
        p2[...] = ps1[:, 128:192]
        p3[...] = ps1[:, 192:256]
        m2_ref[...] = acc_ref[:, 2 * H:] * dis


def _mm_call(x, wcat, h, w1h, dis):
    return pl.pallas_call(
        _mm_body,
        grid=(N // RB, KS),
        in_specs=[
            pl.BlockSpec((RB, KB), lambda i, k: (i, k)),
            pl.BlockSpec((KB, 3 * H), lambda i, k: (k, 0)),
            pl.BlockSpec((RB, H), lambda i, k: (i, 0)),
            pl.BlockSpec((H, 2 * H), lambda i, k: (0, 0)),
            pl.BlockSpec((RB, 1), lambda i, k: (i, 0)),
        ],
        out_specs=[pl.BlockSpec((RB, 64), lambda i, k: (i, 0))] * 4
        + [pl.BlockSpec((RB, H), lambda i, k: (i, 0))],
        out_shape=[jax.ShapeDtypeStruct((N, 64), jnp.float32)] * 4
        + [jax.ShapeDtypeStruct((N, H), jnp.float32)],
        scratch_shapes=[pltpu.VMEM((RB, 3 * H), jnp.float32)],
        compiler_params=pltpu.CompilerParams(
            dimension_semantics=("parallel", "arbitrary")),
    )(x, wcat, h, w1h, dis)


RG = 512


def _gate1_body(sa0, sa1, sa2, sa3, sb0, sb1, sb2, sb3,
                p0, p1, p2, p3, dis_ref, b1_ref, ru_ref):
    dis = dis_ref[...]
    sas = (sa0, sa1, sa2, sa3)
    sbs = (sb0, sb1, sb2, sb3)
    ps = (p0, p1, p2, p3)
    for c in range(4):
        q = (sas[c][...] + sbs[c][...] + ps[c][...]) * dis
        q = q + b1_ref[0, c * 64:(c + 1) * 64][None, :]
        ru_ref[:, c * 64:(c + 1) * 64] = jax.nn.sigmoid(q)


def _gate1_call(s1, ps1, dis, b1r):
    blk = pl.BlockSpec((RG, 64), lambda i: (i, 0))
    blk_hi = pl.BlockSpec((RG, 64), lambda i: (i + N // RG, 0))
    return pl.pallas_call(
        _gate1_body,
        grid=(N // RG,),
        in_specs=[blk] * 4 + [blk_hi] * 4 + [blk] * 4
        + [pl.BlockSpec((RG, 1), lambda i: (i, 0)),
           pl.BlockSpec((1, 2 * H), lambda i: (0, 0))],
        out_specs=pl.BlockSpec((RG, 2 * H), lambda i: (i, 0)),
        out_shape=jax.ShapeDtypeStruct((N, 2 * H), jnp.float32),
        compiler_params=pltpu.CompilerParams(
            dimension_semantics=("parallel",)),
    )(*s1, *s1, *ps1, dis, b1r)


def _mm2_body(r_ref, h_ref, w2h_ref, m2s_ref, dis_ref, q0_ref, q1_ref):
    rh = r_ref[...] * h_ref[...]
    prod = jnp.dot(rh, w2h_ref[...], preferred_element_type=jnp.float32)
    ps2 = m2s_ref[...] + prod * dis_ref[...]
    q0_ref[...] = ps2[:, :64]
    q1_ref[...] = ps2[:, 64:]


def _mm2_call(r, h, w2h, m2s, dis):
    return pl.pallas_call(
        _mm2_body,
        grid=(N // RG,),
        in_specs=[
            pl.BlockSpec((RG, H), lambda i: (i, 0)),
            pl.BlockSpec((RG, H), lambda i: (i, 0)),
            pl.BlockSpec((H, H), lambda i: (0, 0)),
            pl.BlockSpec((RG, H), lambda i: (i, 0)),
            pl.BlockSpec((RG, 1), lambda i: (i, 0)),
        ],
        out_specs=[pl.BlockSpec((RG, 64), lambda i: (i, 0))] * 2,
        out_shape=[jax.ShapeDtypeStruct((N, 64), jnp.float32)] * 2,
        compiler_params=pltpu.CompilerParams(
            dimension_semantics=("parallel",)),
    )(r, h, w2h, m2s, dis)


def _gate2_body(sa0, sa1, sb0, sb1, p0, p1, dis_ref, b2_ref,
                u_ref, h_ref, out_ref):
    dis = dis_ref[...]
    u = u_ref[...]
    h = h_ref[...]
    sas = (sa0, sa1)
    sbs = (sb0, sb1)
    ps = (p0, p1)
    for c in range(2):
        q = (sas[c][...] + sbs[c][...] + ps[c][...]) * dis
        q = q + b2_ref[0, c * 64:(c + 1) * 64][None, :]
        cv = jnp.tanh(q)
        lo, hi = c * 64, (c + 1) * 64
        out_ref[:, lo:hi] = u[:, lo:hi] * h[:, lo:hi] + (1.0 - u[:, lo:hi]) * cv


def _gate2_call(s2, ps2, dis, b2r, u, h):
    blk = pl.BlockSpec((RG, 64), lambda i: (i, 0))
    blk_hi = pl.BlockSpec((RG, 64), lambda i: (i + N // RG, 0))
    blkh = pl.BlockSpec((RG, H), lambda i: (i, 0))
    return pl.pallas_call(
        _gate2_body,
        grid=(N // RG,),
        in_specs=[blk] * 2 + [blk_hi] * 2 + [blk] * 2
        + [pl.BlockSpec((RG, 1), lambda i: (i, 0)),
           pl.BlockSpec((1, H), lambda i: (0, 0)),
           blkh, blkh],
        out_specs=pl.BlockSpec((RG, H), lambda i: (i, 0)),
        out_shape=jax.ShapeDtypeStruct((N, H), jnp.float32),
        compiler_params=pltpu.CompilerParams(
            dimension_semantics=("parallel",)),
    )(*s2, *s2, *ps2, dis, b2r, u, h)


_deg_call = _make_deg_kernel()
_scatter4_call = _make_scatter_kernel(4)
_scatter2_call = _make_scatter_kernel(2)


def kernel(x, edge_index, hidden_state, W1, b1, W2, b2):
    src = edge_index[0]
    dst = edge_index[1]
    W1x, W1h = W1[:F], W1[F:]
    W2x, W2h = W2[:F], W2[F:]
    wcat = jnp.concatenate([W1x, W2x], axis=1).astype(jnp.bfloat16)
    zeros_n = jnp.zeros((N,), jnp.float32)
    zrows = jnp.zeros((N // NS, CC), jnp.float32)

    degp = _deg_call(dst, zeros_n)
    deg = 1.0 + degp[:N] + degp[N:]
    dis = lax.rsqrt(deg).reshape(N, 1)

    ps1 = _mm_call(x, wcat, hidden_state, W1h, dis)
    ps1c, m2s = ps1[:4], ps1[4]

    s1 = _scatter4_call(*ps1c, src, dst, zrows)

    ru = _gate1_call(s1, ps1c, dis, b1.reshape(1, 2 * H))

    ru3 = ru.reshape(N // F, 2, (F // 2) * 2 * H)
    r = ru3[:, 0].reshape(N, H)
    u = ru3[:, 1].reshape(N, H)

    ps2c = _mm2_call(r, hidden_state, W2h, m2s, dis)
    s2 = _scatter2_call(*ps2c, src, dst, zrows)

    return _gate2_call(s2, ps2c, dis, b2.reshape(1, H), u, hidden_state)


# speedup vs baseline: 1.9493x; 1.0999x over previous
import functools

import jax
import jax.numpy as jnp
from jax import lax
from jax.experimental import pallas as pl
from jax.experimental.pallas import tpu as pltpu
from jax.experimental.pallas import tpu_sc as plsc

H = 128
F = 4096
N = 16384
E = 262144
NC = 2
NS = 16
NW = NC * NS

CC = 64
BB = 256


def _sc_mesh():
    return plsc.VectorSubcoreMesh(core_axis_name="c", subcore_axis_name="s")


def _make_deg_kernel():
    EPW = E // NW
    RR = N // NS

    @functools.partial(
        pl.kernel,
        mesh=_sc_mesh(),
        out_type=jax.ShapeDtypeStruct((NC * N,), jnp.float32),
        scratch_types=[
            pltpu.VMEM((EPW,), jnp.int32),
            pltpu.VMEM((N,), jnp.float32),
            pltpu.VMEM((NS, RR), jnp.float32),
            pltpu.VMEM((RR,), jnp.float32),
            pltpu.VMEM_SHARED((NS, N), jnp.float32),
        ],
        compiler_params=pltpu.CompilerParams(needs_layout_passes=False),
    )
    def deg_kernel(dst_hbm, zeros_hbm, out_hbm, didx, hist, tmp16, accv, stage):
        cid = lax.axis_index("c")
        sid = lax.axis_index("s")
        wid = cid * NS + sid
        pltpu.sync_copy(zeros_hbm, hist)
        pltpu.sync_copy(dst_hbm.at[pl.ds(wid * EPW, EPW)], didx)
        ones = jnp.ones((16,), jnp.float32)

        def hbody(j, carry):
            dvec = didx[pl.ds(j * 16, 16)]
            plsc.addupdate_scatter(hist, [dvec], ones)
            return carry

        lax.fori_loop(0, EPW // 16, hbody, 0)
        pltpu.sync_copy(hist, stage.at[sid])
        plsc.subcore_barrier()
        pltpu.sync_copy(stage.at[:, pl.ds(sid * RR, RR)], tmp16)

        def rbody(j, carry):
            s = tmp16[0, pl.ds(j * 16, 16)]
            for k in range(1, NS):
                s = s + tmp16[k, pl.ds(j * 16, 16)]
            accv[pl.ds(j * 16, 16)] = s
            return carry

        lax.fori_loop(0, RR // 16, rbody, 0)
        pltpu.sync_copy(accv, out_hbm.at[pl.ds(cid * N + sid * RR, RR)])

    return deg_kernel


def _make_scatter_kernel(nchunk):
    EPW = E // NW
    RZ = N // NS
    NB = EPW // BB

    @functools.partial(
        pl.kernel,
        mesh=_sc_mesh(),
        out_type=tuple(
            jax.ShapeDtypeStruct((NC * N, CC), jnp.float32)
            for _ in range(nchunk)
        ),
        scratch_types=[
            pltpu.VMEM((EPW,), jnp.int32),
            pltpu.VMEM((EPW,), jnp.int32),
            pltpu.VMEM((BB,), jnp.int32),
            pltpu.VMEM((BB,), jnp.int32),
            pltpu.VMEM((BB,), jnp.int32),
            pltpu.VMEM((BB,), jnp.int32),
            pltpu.VMEM((BB, CC), jnp.float32),
            pltpu.VMEM((BB, CC), jnp.float32),
            pltpu.VMEM_SHARED((N, CC), jnp.float32),
            pltpu.SemaphoreType.DMA,
            pltpu.SemaphoreType.DMA,
            pltpu.SemaphoreType.DMA,
        ],
        compiler_params=pltpu.CompilerParams(use_tc_tiling_on_sc=False),
    )
    def scatter_kernel(*refs):
        tables = refs[:nchunk]
        src_hbm, dst_hbm, zrows = refs[nchunk:nchunk + 3]
        outs = refs[nchunk + 3:2 * nchunk + 3]
        (sall, dall, sidxa, didxa, sidxb, didxb,
         msga, msgb, acc, sema, semb, sems) = refs[2 * nchunk + 3:]
        cid = lax.axis_index("c")
        sid = lax.axis_index("s")
        ebase = (cid * NS + sid) * EPW
        pltpu.sync_copy(src_hbm.at[pl.ds(ebase, EPW)], sall)
        pltpu.sync_copy(dst_hbm.at[pl.ds(ebase, EPW)], dall)

        def copy_idx(base, sdst, ddst):
            for i in range(BB // 16):
                sdst[pl.ds(i * 16, 16)] = sall[pl.ds(base + i * 16, 16)]
                ddst[pl.ds(i * 16, 16)] = dall[pl.ds(base + i * 16, 16)]

        for c in range(nchunk):
            pltpu.sync_copy(zrows, acc.at[pl.ds(sid * RZ, RZ)])
            plsc.subcore_barrier()

            def body2(t, carry):
                copy_idx(2 * t * BB, sidxa, didxa)
                ga = pltpu.async_copy(tables[c].at[sidxa], msga, sema)
                copy_idx((2 * t + 1) * BB, sidxb, didxb)
                gb = pltpu.async_copy(tables[c].at[sidxb], msgb, semb)
                ga.wait()
                sa = pltpu.async_copy(msga, acc.at[didxa], sems, add=True)
                gb.wait()
                sb = pltpu.async_copy(msgb, acc.at[didxb], sems, add=True)
                sa.wait()
                sb.wait()
                return carry

            lax.fori_loop(0, NB // 2, body2, 0)
            plsc.subcore_barrier()
            pltpu.sync_copy(
                acc.at[pl.ds(sid * RZ, RZ)],
                outs[c].at[pl.ds(cid * N + sid * RZ, RZ)],
            )

    return scatter_kernel


RB = 1024
KB = 512
KS = F // KB


def _mm_body(x_ref, w_ref, h_ref, w1h_ref, dis_ref,
             p0, p1, p2, p3, m2_ref, acc_ref):
    k = pl.program_id(1)

    @pl.when(k == 0)
    def _():
        acc_ref[...] = jnp.zeros_like(acc_ref)

    acc_ref[...] += jnp.dot(x_ref[...].astype(jnp.bfloat16), w_ref[...],
                            preferred_element_type=jnp.float32)

    @pl.when(k == KS - 1)
    def _():
        dis = dis_ref[...]
        m1 = acc_ref[:, :2 * H] + jnp.dot(h_ref[...], w1h_ref[...],
                                          preferred_element_type=jnp.float32)
        ps1 = m1 * dis
        p0[...] = ps1[:, 0:64]
        p1[...] = ps1[:, 64:128]
        p2[...] = ps1[:, 128:192]
        p3[...] = ps1[:, 192:256]
        m2_ref[...] = acc_ref[:, 2 * H:] * dis


def _mm_call(x, wcat, h, w1h, dis):
    return pl.pallas_call(
        _mm_body,
        grid=(N // RB, KS),
        in_specs=[
            pl.BlockSpec((RB, KB), lambda i, k: (i, k)),
            pl.BlockSpec((KB, 3 * H), lambda i, k: (k, 0)),
            pl.BlockSpec((RB, H), lambda i, k: (i, 0)),
            pl.BlockSpec((H, 2 * H), lambda i, k: (0, 0)),
            pl.BlockSpec((RB, 1), lambda i, k: (i, 0)),
        ],
        out_specs=[pl.BlockSpec((RB, 64), lambda i, k: (i, 0))] * 4
        + [pl.BlockSpec((RB, H), lambda i, k: (i, 0))],
        out_shape=[jax.ShapeDtypeStruct((N, 64), jnp.float32)] * 4
        + [jax.ShapeDtypeStruct((N, H), jnp.float32)],
        scratch_shapes=[pltpu.VMEM((RB, 3 * H), jnp.float32)],
        compiler_params=pltpu.CompilerParams(
            dimension_semantics=("parallel", "arbitrary")),
    )(x, wcat, h, w1h, dis)


RG = 512


def _gate1_body(sa0, sa1, sa2, sa3, sb0, sb1, sb2, sb3,
                p0, p1, p2, p3, dis_ref, b1_ref, ru_ref):
    dis = dis_ref[...]
    sas = (sa0, sa1, sa2, sa3)
    sbs = (sb0, sb1, sb2, sb3)
    ps = (p0, p1, p2, p3)
    for c in range(4):
        q = (sas[c][...] + sbs[c][...] + ps[c][...]) * dis
        q = q + b1_ref[0, c * 64:(c + 1) * 64][None, :]
        ru_ref[:, c * 64:(c + 1) * 64] = jax.nn.sigmoid(q)


def _gate1_call(s1, ps1, dis, b1r):
    blk = pl.BlockSpec((RG, 64), lambda i: (i, 0))
    blk_hi = pl.BlockSpec((RG, 64), lambda i: (i + N // RG, 0))
    return pl.pallas_call(
        _gate1_body,
        grid=(N // RG,),
        in_specs=[blk] * 4 + [blk_hi] * 4 + [blk] * 4
        + [pl.BlockSpec((RG, 1), lambda i: (i, 0)),
           pl.BlockSpec((1, 2 * H), lambda i: (0, 0))],
        out_specs=pl.BlockSpec((RG, 2 * H), lambda i: (i, 0)),
        out_shape=jax.ShapeDtypeStruct((N, 2 * H), jnp.float32),
        compiler_params=pltpu.CompilerParams(
            dimension_semantics=("parallel",)),
    )(*s1, *s1, *ps1, dis, b1r)


def _mm2_body(r_ref, h_ref, w2h_ref, m2s_ref, dis_ref, q0_ref, q1_ref):
    rh = r_ref[...] * h_ref[...]
    prod = jnp.dot(rh, w2h_ref[...], preferred_element_type=jnp.float32)
    ps2 = m2s_ref[...] + prod * dis_ref[...]
    q0_ref[...] = ps2[:, :64]
    q1_ref[...] = ps2[:, 64:]


def _mm2_call(r, h, w2h, m2s, dis):
    return pl.pallas_call(
        _mm2_body,
        grid=(N // RG,),
        in_specs=[
            pl.BlockSpec((RG, H), lambda i: (i, 0)),
            pl.BlockSpec((RG, H), lambda i: (i, 0)),
            pl.BlockSpec((H, H), lambda i: (0, 0)),
            pl.BlockSpec((RG, H), lambda i: (i, 0)),
            pl.BlockSpec((RG, 1), lambda i: (i, 0)),
        ],
        out_specs=[pl.BlockSpec((RG, 64), lambda i: (i, 0))] * 2,
        out_shape=[jax.ShapeDtypeStruct((N, 64), jnp.float32)] * 2,
        compiler_params=pltpu.CompilerParams(
            dimension_semantics=("parallel",)),
    )(r, h, w2h, m2s, dis)


def _gate2_body(sa0, sa1, sb0, sb1, p0, p1, dis_ref, b2_ref,
                u_ref, h_ref, out_ref):
    dis = dis_ref[...]
    u = u_ref[...]
    h = h_ref[...]
    sas = (sa0, sa1)
    sbs = (sb0, sb1)
    ps = (p0, p1)
    for c in range(2):
        q = (sas[c][...] + sbs[c][...] + ps[c][...]) * dis
        q = q + b2_ref[0, c * 64:(c + 1) * 64][None, :]
        cv = jnp.tanh(q)
        lo, hi = c * 64, (c + 1) * 64
        out_ref[:, lo:hi] = u[:, lo:hi] * h[:, lo:hi] + (1.0 - u[:, lo:hi]) * cv


def _gate2_call(s2, ps2, dis, b2r, u, h):
    blk = pl.BlockSpec((RG, 64), lambda i: (i, 0))
    blk_hi = pl.BlockSpec((RG, 64), lambda i: (i + N // RG, 0))
    blkh = pl.BlockSpec((RG, H), lambda i: (i, 0))
    return pl.pallas_call(
        _gate2_body,
        grid=(N // RG,),
        in_specs=[blk] * 2 + [blk_hi] * 2 + [blk] * 2
        + [pl.BlockSpec((RG, 1), lambda i: (i, 0)),
           pl.BlockSpec((1, H), lambda i: (0, 0)),
           blkh, blkh],
        out_specs=pl.BlockSpec((RG, H), lambda i: (i, 0)),
        out_shape=jax.ShapeDtypeStruct((N, H), jnp.float32),
        compiler_params=pltpu.CompilerParams(
            dimension_semantics=("parallel",)),
    )(*s2, *s2, *ps2, dis, b2r, u, h)


_deg_call = _make_deg_kernel()
_scatter4_call = _make_scatter_kernel(4)
_scatter2_call = _make_scatter_kernel(2)


def kernel(x, edge_index, hidden_state, W1, b1, W2, b2):
    src = edge_index[0]
    dst = edge_index[1]
    W1x, W1h = W1[:F], W1[F:]
    W2x, W2h = W2[:F], W2[F:]
    wcat = jnp.concatenate([W1x, W2x], axis=1).astype(jnp.bfloat16)
    zeros_n = jnp.zeros((N,), jnp.float32)
    zrows = jnp.zeros((N // NS, CC), jnp.float32)

    degp = _deg_call(dst, zeros_n)
    deg = 1.0 + degp[:N] + degp[N:]
    dis = lax.rsqrt(deg).reshape(N, 1)

    ps1 = _mm_call(x, wcat, hidden_state, W1h, dis)
    ps1c, m2s = ps1[:4], ps1[4]

    s1 = _scatter4_call(*ps1c, src, dst, zrows)

    ru = _gate1_call(s1, ps1c, dis, b1.reshape(1, 2 * H))

    ru3 = ru.reshape(N // F, 2, (F // 2) * 2 * H)
    r = ru3[:, 0].reshape(N, H)
    u = ru3[:, 1].reshape(N, H)

    ps2c = _mm2_call(r, hidden_state, W2h, m2s, dis)
    s2 = _scatter2_call(*ps2c, src, dst, zrows)

    return _gate2_call(s2, ps2c, dis, b2.reshape(1, H), u, hidden_state)


# speedup vs baseline: 2.0483x; 1.0508x over previous
import functools

import jax
import jax.numpy as jnp
from jax import lax
from jax.experimental import pallas as pl
from jax.experimental.pallas import tpu as pltpu
from jax.experimental.pallas import tpu_sc as plsc

H = 128
F = 4096
N = 16384
E = 262144
NC = 2
NS = 16
NW = NC * NS

CC = 64
BB = 256


def _sc_mesh():
    return plsc.VectorSubcoreMesh(core_axis_name="c", subcore_axis_name="s")


def _make_deg_kernel():
    EPW = E // NW
    RR = N // NS

    @functools.partial(
        pl.kernel,
        mesh=_sc_mesh(),
        out_type=jax.ShapeDtypeStruct((NC * N,), jnp.float32),
        scratch_types=[
            pltpu.VMEM((EPW,), jnp.int32),
            pltpu.VMEM((N,), jnp.float32),
            pltpu.VMEM((NS, RR), jnp.float32),
            pltpu.VMEM((RR,), jnp.float32),
            pltpu.VMEM_SHARED((NS, N), jnp.float32),
        ],
        compiler_params=pltpu.CompilerParams(needs_layout_passes=False),
    )
    def deg_kernel(dst_hbm, zeros_hbm, out_hbm, didx, hist, tmp16, accv, stage):
        cid = lax.axis_index("c")
        sid = lax.axis_index("s")
        wid = cid * NS + sid
        pltpu.sync_copy(zeros_hbm, hist)
        pltpu.sync_copy(dst_hbm.at[pl.ds(wid * EPW, EPW)], didx)
        ones = jnp.ones((16,), jnp.float32)

        def hbody(j, carry):
            dvec = didx[pl.ds(j * 16, 16)]
            plsc.addupdate_scatter(hist, [dvec], ones)
            return carry

        lax.fori_loop(0, EPW // 16, hbody, 0)
        pltpu.sync_copy(hist, stage.at[sid])
        plsc.subcore_barrier()
        pltpu.sync_copy(stage.at[:, pl.ds(sid * RR, RR)], tmp16)

        def rbody(j, carry):
            s = tmp16[0, pl.ds(j * 16, 16)]
            for k in range(1, NS):
                s = s + tmp16[k, pl.ds(j * 16, 16)]
            accv[pl.ds(j * 16, 16)] = s
            return carry

        lax.fori_loop(0, RR // 16, rbody, 0)
        pltpu.sync_copy(accv, out_hbm.at[pl.ds(cid * N + sid * RR, RR)])

    return deg_kernel


def _make_scatter_kernel(nchunk):
    EPW = E // NW
    RZ = N // NS
    NB = EPW // BB

    @functools.partial(
        pl.kernel,
        mesh=_sc_mesh(),
        out_type=tuple(
            jax.ShapeDtypeStruct((NC * N, CC), jnp.float32)
            for _ in range(nchunk)
        ),
        scratch_types=[
            pltpu.VMEM((EPW,), jnp.int32),
            pltpu.VMEM((EPW,), jnp.int32),
            pltpu.VMEM((BB,), jnp.int32),
            pltpu.VMEM((BB,), jnp.int32),
            pltpu.VMEM((BB,), jnp.int32),
            pltpu.VMEM((BB,), jnp.int32),
            pltpu.VMEM((BB, CC), jnp.float32),
            pltpu.VMEM((BB, CC), jnp.float32),
            pltpu.VMEM_SHARED((N, CC), jnp.float32),
            pltpu.SemaphoreType.DMA,
            pltpu.SemaphoreType.DMA,
            pltpu.SemaphoreType.DMA,
        ],
        compiler_params=pltpu.CompilerParams(use_tc_tiling_on_sc=False),
    )
    def scatter_kernel(*refs):
        tables = refs[:nchunk]
        src_hbm, dst_hbm, zrows = refs[nchunk:nchunk + 3]
        outs = refs[nchunk + 3:2 * nchunk + 3]
        (sall, dall, sidxa, didxa, sidxb, didxb,
         msga, msgb, acc, sema, semb, sems) = refs[2 * nchunk + 3:]
        cid = lax.axis_index("c")
        sid = lax.axis_index("s")
        ebase = (cid * NS + sid) * EPW
        pltpu.sync_copy(src_hbm.at[pl.ds(ebase, EPW)], sall)
        pltpu.sync_copy(dst_hbm.at[pl.ds(ebase, EPW)], dall)

        def copy_idx(base, sdst, ddst):
            for i in range(BB // 16):
                sdst[pl.ds(i * 16, 16)] = sall[pl.ds(base + i * 16, 16)]
                ddst[pl.ds(i * 16, 16)] = dall[pl.ds(base + i * 16, 16)]

        for c in range(nchunk):
            pltpu.sync_copy(zrows, acc.at[pl.ds(sid * RZ, RZ)])
            plsc.subcore_barrier()

            def body2(t, carry):
                copy_idx(2 * t * BB, sidxa, didxa)
                ga = pltpu.async_copy(tables[c].at[sidxa], msga, sema)
                copy_idx((2 * t + 1) * BB, sidxb, didxb)
                gb = pltpu.async_copy(tables[c].at[sidxb], msgb, semb)
                ga.wait()
                sa = pltpu.async_copy(msga, acc.at[didxa], sems, add=True)
                gb.wait()
                sb = pltpu.async_copy(msgb, acc.at[didxb], sems, add=True)
                sa.wait()
                sb.wait()
                return carry

            lax.fori_loop(0, NB // 2, body2, 0)
            plsc.subcore_barrier()
            pltpu.sync_copy(
                acc.at[pl.ds(sid * RZ, RZ)],
                outs[c].at[pl.ds(cid * N + sid * RZ, RZ)],
            )

    return scatter_kernel


RB = 2048
KB = 512
KS = F // KB


def _mm_body(x_ref, w_ref, h_ref, w1h_ref, dis_ref,
             p0, p1, p2, p3, m2_ref, acc_ref):
    k = pl.program_id(1)

    @pl.when(k == 0)
    def _():
        acc_ref[...] = jnp.zeros_like(acc_ref)

    acc_ref[...] += jnp.dot(x_ref[...].astype(jnp.bfloat16), w_ref[...],
                            preferred_element_type=jnp.float32)

    @pl.when(k == KS - 1)
    def _():
        dis = dis_ref[...]
        m1 = acc_ref[:, :2 * H] + jnp.dot(h_ref[...], w1h_ref[...],
                                          preferred_element_type=jnp.float32)
        ps1 = m1 * dis
        p0[...] = ps1[:, 0:64]
        p1[...] = ps1[:, 64:128]
        p2[...] = ps1[:, 128:192]
        p3[...] = ps1[:, 192:256]
        m2_ref[...] = acc_ref[:, 2 * H:] * dis


def _mm_call(x, wcat, h, w1h, dis):
    return pl.pallas_call(
        _mm_body,
        grid=(N // RB, KS),
        in_specs=[
            pl.BlockSpec((RB, KB), lambda i, k: (i, k)),
            pl.BlockSpec((KB, 3 * H), lambda i, k: (k, 0)),
            pl.BlockSpec((RB, H), lambda i, k: (i, 0)),
            pl.BlockSpec((H, 2 * H), lambda i, k: (0, 0)),
            pl.BlockSpec((RB, 1), lambda i, k: (i, 0)),
        ],
        out_specs=[pl.BlockSpec((RB, 64), lambda i, k: (i, 0))] * 4
        + [pl.BlockSpec((RB, H), lambda i, k: (i, 0))],
        out_shape=[jax.ShapeDtypeStruct((N, 64), jnp.float32)] * 4
        + [jax.ShapeDtypeStruct((N, H), jnp.float32)],
        scratch_shapes=[pltpu.VMEM((RB, 3 * H), jnp.float32)],
        compiler_params=pltpu.CompilerParams(
            dimension_semantics=("parallel", "arbitrary")),
    )(x, wcat, h, w1h, dis)


RG = 512


def _gate1_body(sa0, sa1, sa2, sa3, sb0, sb1, sb2, sb3,
                p0, p1, p2, p3, dis_ref, b1_ref, ru_ref):
    dis = dis_ref[...]
    sas = (sa0, sa1, sa2, sa3)
    sbs = (sb0, sb1, sb2, sb3)
    ps = (p0, p1, p2, p3)
    for c in range(4):
        q = (sas[c][...] + sbs[c][...] + ps[c][...]) * dis
        q = q + b1_ref[0, c * 64:(c + 1) * 64][None, :]
        ru_ref[:, c * 64:(c + 1) * 64] = jax.nn.sigmoid(q)


def _gate1_call(s1, ps1, dis, b1r):
    blk = pl.BlockSpec((RG, 64), lambda i: (i, 0))
    blk_hi = pl.BlockSpec((RG, 64), lambda i: (i + N // RG, 0))
    return pl.pallas_call(
        _gate1_body,
        grid=(N // RG,),
        in_specs=[blk] * 4 + [blk_hi] * 4 + [blk] * 4
        + [pl.BlockSpec((RG, 1), lambda i: (i, 0)),
           pl.BlockSpec((1, 2 * H), lambda i: (0, 0))],
        out_specs=pl.BlockSpec((RG, 2 * H), lambda i: (i, 0)),
        out_shape=jax.ShapeDtypeStruct((N, 2 * H), jnp.float32),
        compiler_params=pltpu.CompilerParams(
            dimension_semantics=("parallel",)),
    )(*s1, *s1, *ps1, dis, b1r)


def _mm2_body(r_ref, h_ref, w2h_ref, m2s_ref, dis_ref, q0_ref, q1_ref):
    rh = r_ref[...] * h_ref[...]
    prod = jnp.dot(rh, w2h_ref[...], preferred_element_type=jnp.float32)
    ps2 = m2s_ref[...] + prod * dis_ref[...]
    q0_ref[...] = ps2[:, :64]
    q1_ref[...] = ps2[:, 64:]


def _mm2_call(r, h, w2h, m2s, dis):
    return pl.pallas_call(
        _mm2_body,
        grid=(N // RG,),
        in_specs=[
            pl.BlockSpec((RG, H), lambda i: (i, 0)),
            pl.BlockSpec((RG, H), lambda i: (i, 0)),
            pl.BlockSpec((H, H), lambda i: (0, 0)),
            pl.BlockSpec((RG, H), lambda i: (i, 0)),
            pl.BlockSpec((RG, 1), lambda i: (i, 0)),
        ],
        out_specs=[pl.BlockSpec((RG, 64), lambda i: (i, 0))] * 2,
        out_shape=[jax.ShapeDtypeStruct((N, 64), jnp.float32)] * 2,
        compiler_params=pltpu.CompilerParams(
            dimension_semantics=("parallel",)),
    )(r, h, w2h, m2s, dis)


def _gate2_body(sa0, sa1, sb0, sb1, p0, p1, dis_ref, b2_ref,
                u_ref, h_ref, out_ref):
    dis = dis_ref[...]
    u = u_ref[...]
    h = h_ref[...]
    sas = (sa0, sa1)
    sbs = (sb0, sb1)
    ps = (p0, p1)
    for c in range(2):
        q = (sas[c][...] + sbs[c][...] + ps[c][...]) * dis
        q = q + b2_ref[0, c * 64:(c + 1) * 64][None, :]
        cv = jnp.tanh(q)
        lo, hi = c * 64, (c + 1) * 64
        out_ref[:, lo:hi] = u[:, lo:hi] * h[:, lo:hi] + (1.0 - u[:, lo:hi]) * cv


def _gate2_call(s2, ps2, dis, b2r, u, h):
    blk = pl.BlockSpec((RG, 64), lambda i: (i, 0))
    blk_hi = pl.BlockSpec((RG, 64), lambda i: (i + N // RG, 0))
    blkh = pl.BlockSpec((RG, H), lambda i: (i, 0))
    return pl.pallas_call(
        _gate2_body,
        grid=(N // RG,),
        in_specs=[blk] * 2 + [blk_hi] * 2 + [blk] * 2
        + [pl.BlockSpec((RG, 1), lambda i: (i, 0)),
           pl.BlockSpec((1, H), lambda i: (0, 0)),
           blkh, blkh],
        out_specs=pl.BlockSpec((RG, H), lambda i: (i, 0)),
        out_shape=jax.ShapeDtypeStruct((N, H), jnp.float32),
        compiler_params=pltpu.CompilerParams(
            dimension_semantics=("parallel",)),
    )(*s2, *s2, *ps2, dis, b2r, u, h)


_deg_call = _make_deg_kernel()
_scatter4_call = _make_scatter_kernel(4)
_scatter2_call = _make_scatter_kernel(2)


def kernel(x, edge_index, hidden_state, W1, b1, W2, b2):
    src = edge_index[0]
    dst = edge_index[1]
    W1x, W1h = W1[:F], W1[F:]
    W2x, W2h = W2[:F], W2[F:]
    wcat = jnp.concatenate([W1x, W2x], axis=1).astype(jnp.bfloat16)
    zeros_n = jnp.zeros((N,), jnp.float32)
    zrows = jnp.zeros((N // NS, CC), jnp.float32)

    degp = _deg_call(dst, zeros_n)
    deg = 1.0 + degp[:N] + degp[N:]
    dis = lax.rsqrt(deg).reshape(N, 1)

    ps1 = _mm_call(x, wcat, hidden_state, W1h, dis)
    ps1c, m2s = ps1[:4], ps1[4]

    s1 = _scatter4_call(*ps1c, src, dst, zrows)

    ru = _gate1_call(s1, ps1c, dis, b1.reshape(1, 2 * H))

    ru3 = ru.reshape(N // F, 2, (F // 2) * 2 * H)
    r = ru3[:, 0].reshape(N, H)
    u = ru3[:, 1].reshape(N, H)

    ps2c = _mm2_call(r, hidden_state, W2h, m2s, dis)
    s2 = _scatter2_call(*ps2c, src, dst, zrows)

    return _gate2_call(s2, ps2c, dis, b2.reshape(1, H), u, hidden_state)
